# baseline (speedup 1.0000x reference)
import jax
import jax.numpy as jnp
from jax.experimental import pallas as pl

SEQ_LEN = 200
EMBED_DIM = 64
LANES = 128
SPAIR = SEQ_LEN // 2
B_BLK = 64


def _body(x_ref, pe_ref, o_ref):
    xb = x_ref[...]
    pe = pe_ref[...]
    a = xb[:, :, :EMBED_DIM]
    b = xb[:, :, EMBED_DIM:]
    ka = (jnp.sum(a, axis=2, keepdims=True) != 0.0).astype(xb.dtype)
    kb = (jnp.sum(b, axis=2, keepdims=True) != 0.0).astype(xb.dtype)
    o_ref[:, :, :EMBED_DIM] = a + ka * pe[None, :, :EMBED_DIM]
    o_ref[:, :, EMBED_DIM:] = b + kb * pe[None, :, EMBED_DIM:]


def kernel(x, table):
    B, S, E = x.shape
    x3 = x.reshape(B, SPAIR, LANES)
    pe = table[1:, :].reshape(SPAIR, LANES)
    grid = (B // B_BLK,)
    out = pl.pallas_call(
        _body,
        grid=grid,
        in_specs=[
            pl.BlockSpec((B_BLK, SPAIR, LANES), lambda i: (i, 0, 0)),
            pl.BlockSpec((SPAIR, LANES), lambda i: (0, 0)),
        ],
        out_specs=pl.BlockSpec((B_BLK, SPAIR, LANES), lambda i: (i, 0, 0)),
        out_shape=jax.ShapeDtypeStruct((B, SPAIR, LANES), x.dtype),
    )(x3, pe)
    return out.reshape(x.shape)


# speedup vs baseline: 1.0359x; 1.0359x over previous
import jax
import jax.numpy as jnp
from jax.experimental import pallas as pl

SEQ_LEN = 200
EMBED_DIM = 64
LANES = 128
SPAIR = SEQ_LEN // 2
B_BLK = 128


def _body(x_ref, pe_ref, o_ref):
    xb = x_ref[...]
    pe = pe_ref[...]
    a = xb[:, :, :EMBED_DIM]
    b = xb[:, :, EMBED_DIM:]
    ka = (jnp.sum(a, axis=2, keepdims=True) != 0.0).astype(xb.dtype)
    kb = (jnp.sum(b, axis=2, keepdims=True) != 0.0).astype(xb.dtype)
    o_ref[:, :, :EMBED_DIM] = a + ka * pe[None, :, :EMBED_DIM]
    o_ref[:, :, EMBED_DIM:] = b + kb * pe[None, :, EMBED_DIM:]


def kernel(x, table):
    B, S, E = x.shape
    x3 = x.reshape(B, SPAIR, LANES)
    pe = table[1:, :].reshape(SPAIR, LANES)
    grid = (B // B_BLK,)
    out = pl.pallas_call(
        _body,
        grid=grid,
        in_specs=[
            pl.BlockSpec((B_BLK, SPAIR, LANES), lambda i: (i, 0, 0)),
            pl.BlockSpec((SPAIR, LANES), lambda i: (0, 0)),
        ],
        out_specs=pl.BlockSpec((B_BLK, SPAIR, LANES), lambda i: (i, 0, 0)),
        out_shape=jax.ShapeDtypeStruct((B, SPAIR, LANES), x.dtype),
    )(x3, pe)
    return out.reshape(x.shape)
